# Initial kernel scaffold; baseline (speedup 1.0000x reference)
import jax
import jax.numpy as jnp
from jax.experimental import pallas as pl


def kernel(pha, amp):
    raise NotImplementedError("write your pallas kernel here")


# speedup vs baseline: 7.3121x; 7.3121x over previous
import numpy as np
import jax
import jax.numpy as jnp
from jax import lax
from jax.experimental import pallas as pl

N_BINS = 18
B = 32
T = 16384


def _mi_body(cut_ref, pha_ref, amp_ref, out_ref):
    amp = amp_ref[...]
    cut = cut_ref[...]
    lows = cut[0, :N_BINS].reshape(N_BINS, 1)
    highs = cut[0, 1 : N_BINS + 1].reshape(N_BINS, 1)
    inv_log_n = np.float32(1.0 / np.log(float(N_BINS)))
    inv_t = np.float32(1.0 / T)

    def per_j(j, carry):
        pha_j = pha_ref[pl.ds(j, 1), :]
        oh = ((lows < pha_j) & (pha_j < highs)).astype(jnp.float32)
        sums = lax.dot_general(
            amp, oh, (((1,), (1,)), ((), ())),
            preferred_element_type=jnp.float32,
        )
        means = sums * inv_t
        total = jnp.sum(means, axis=1, keepdims=True)
        probs = means / total
        mi = 1.0 + inv_log_n * jnp.sum(probs * jnp.log(probs), axis=1)
        out_ref[pl.ds(j, 1), :] = mi.reshape(1, B)
        return carry

    lax.fori_loop(0, B, per_j, 0)


@jax.jit
def kernel(pha, amp):
    cutoffs = jnp.linspace(-np.pi, np.pi, N_BINS + 1).astype(pha.dtype)
    cutoffs = cutoffs.reshape(1, N_BINS + 1)
    mit = pl.pallas_call(
        _mi_body,
        out_shape=jax.ShapeDtypeStruct((B, B), jnp.float32),
    )(cutoffs, pha, amp)
    return mit.T
